# baseline (speedup 1.0000x reference)
import functools

import jax
import jax.numpy as jnp
from jax import lax
from jax.experimental import pallas as pl
from jax.experimental.pallas import tpu as pltpu
from jax.experimental.pallas import tpu_sc as plsc


def _sc_gather(idx, table):
    (B,) = idx.shape
    _, D = table.shape
    info = plsc.get_sparse_core_info()
    nc, ns = info.num_cores, info.num_subcores
    nw = nc * ns
    b_per_w = B // nw

    mesh = plsc.VectorSubcoreMesh(core_axis_name="c", subcore_axis_name="s")

    @functools.partial(
        pl.kernel,
        mesh=mesh,
        compiler_params=pltpu.CompilerParams(use_tc_tiling_on_sc=False),
        out_type=jax.ShapeDtypeStruct((B, D), jnp.float32),
        scratch_types=[
            pltpu.VMEM((b_per_w,), jnp.int32),
            pltpu.VMEM((b_per_w, D), jnp.float32),
            pltpu.SemaphoreType.DMA,
        ],
    )
    def gather_kernel(idx_hbm, table_hbm, out_hbm, idx_v, rows_v, sem):
        wid = lax.axis_index("s") * nc + lax.axis_index("c")
        base = wid * b_per_w
        pltpu.sync_copy(idx_hbm.at[pl.ds(base, b_per_w)], idx_v)
        pltpu.async_copy(table_hbm.at[idx_v], rows_v, sem).wait()
        pltpu.sync_copy(rows_v, out_hbm.at[pl.ds(base, b_per_w)])

    return gather_kernel(idx, table)


_V_BLK = 512


def _tc_head(tok, pos_rep, W, b2):
    M, D = tok.shape
    V = W.shape[0]
    nv = pl.cdiv(V, _V_BLK)

    def head_kernel(tok_ref, pos_ref, w_ref, b_ref, out_ref):
        h = tok_ref[...] + pos_ref[...]
        acc = lax.dot_general(
            h, w_ref[...], (((1,), (1,)), ((), ())),
            preferred_element_type=jnp.float32,
        )
        out_ref[...] = acc + b_ref[...]

    return pl.pallas_call(
        head_kernel,
        grid=(nv,),
        in_specs=[
            pl.BlockSpec((M, D), lambda j: (0, 0)),
            pl.BlockSpec((M, D), lambda j: (0, 0)),
            pl.BlockSpec((_V_BLK, D), lambda j: (j, 0)),
            pl.BlockSpec((1, _V_BLK), lambda j: (0, j)),
        ],
        out_specs=pl.BlockSpec((M, _V_BLK), lambda j: (0, j)),
        out_shape=jax.ShapeDtypeStruct((M, V), jnp.float32),
    )(tok, pos_rep, W, b2)


def kernel(x, token_table, pos_table, W, b):
    B, T = x.shape
    idx = x.reshape(-1).astype(jnp.int32)
    tok = _sc_gather(idx, token_table)
    pos_rep = jnp.tile(pos_table, (B, 1))
    out2d = _tc_head(tok, pos_rep, W, b.reshape(1, -1))
    return out2d.reshape(B, T, -1)
